# baseline (speedup 1.0000x reference)
import functools

import jax
import jax.numpy as jnp
from jax import lax
from jax.experimental import pallas as pl
from jax.experimental.pallas import tpu as pltpu
from jax.experimental.pallas import tpu_sc as plsc

NUM_EMBEDDINGS = 1000000
EMBEDDING_DIM = 64
BATCH = 16384
HIST_LEN = 50

NC = 2
NS = 16
NW = NC * NS

T2_ROWS = NUM_EMBEDDINGS // 2
KA_FULL = NUM_EMBEDDINGS // 128
KA_MAIN = (KA_FULL // NW) & ~1
KA_REST = KA_FULL - KA_MAIN * NW
BPW = BATCH // NW
NCHB = HIST_LEN * (BPW // 128)

_mesh = plsc.VectorSubcoreMesh(core_axis_name="c", subcore_axis_name="s")


def _iota16():
    return lax.iota(jnp.int32, 16)


def _worker_id():
    return lax.axis_index("s") * NC + lax.axis_index("c")


def _transpose_pairs(v_ref, s_ref, rows):
    @plsc.parallel_loop(0, rows, unroll=16)
    def _(r0):
        t = _iota16() + r0
        rl = t - jnp.where(t >= rows, rows, 0)
        for m in range(4):
            q16 = _iota16() + 16 * m
            vals0 = plsc.load_gather(v_ref, [q16, 2 * rl])
            plsc.store_scatter(s_ref, [rl, q16], vals0)
            vals1 = plsc.load_gather(v_ref, [q16, 2 * rl + 1])
            plsc.store_scatter(s_ref, [rl, q16 + 64], vals1)


@functools.partial(
    pl.kernel,
    mesh=_mesh,
    out_type=jax.ShapeDtypeStruct((T2_ROWS, 128), jnp.float32),
    scratch_types=(
        [pltpu.VMEM((64, 128), jnp.float32) for _ in range(4)]
        + [pltpu.SemaphoreType.DMA for _ in range(4)]
    ),
    compiler_params=pltpu.CompilerParams(needs_layout_passes=False),
)
def _repack(tabt_hbm, tail2_hbm, t2_hbm, v0, v1, s0, s1, in0, in1, out0, out1):
    w = _worker_id()
    k0 = w * KA_MAIN
    vb, sb, ins, outs = (v0, v1), (s0, s1), (in0, in1), (out0, out1)

    def in_start(k, b):
        off = pl.multiple_of(k * 128, 128)
        pltpu.make_async_copy(
            tabt_hbm.at[:, pl.ds(off, 128)], vb[b], ins[b]
        ).start()

    def in_wait(b):
        pltpu.make_async_copy(
            tabt_hbm.at[:, pl.ds(0, 128)], vb[b], ins[b]
        ).wait()

    def out_start(k, b):
        off = pl.multiple_of(k * 64, 64)
        pltpu.make_async_copy(
            sb[b], t2_hbm.at[pl.ds(off, 64), :], outs[b]
        ).start()

    def out_wait(b):
        pltpu.make_async_copy(
            sb[b], t2_hbm.at[pl.ds(0, 64), :], outs[b]
        ).wait()

    in_start(k0, 0)
    in_start(k0 + 1, 1)

    def group(g, carry):
        for b in range(2):
            k = k0 + 2 * g + b
            in_wait(b)

            @pl.when(g >= 1)
            def _():
                out_wait(b)

            _transpose_pairs(vb[b], sb[b], 64)
            out_start(k, b)

            @pl.when(g < KA_MAIN // 2 - 1)
            def _():
                in_start(k + 2, b)

        return carry

    lax.fori_loop(0, KA_MAIN // 2, group, 0)
    out_wait(0)
    out_wait(1)

    @pl.when(w < KA_REST)
    def _():
        k = KA_MAIN * NW + w
        in_start(k, 0)
        in_wait(0)
        _transpose_pairs(vb[0], sb[0], 64)
        out_start(k, 0)
        out_wait(0)

    @pl.when(w == KA_REST)
    def _():
        pltpu.make_async_copy(tail2_hbm, v0.at[pl.ds(0, 32), :], ins[0]).start()
        pltpu.make_async_copy(tail2_hbm, v0.at[pl.ds(0, 32), :], ins[0]).wait()
        pltpu.make_async_copy(
            v0.at[pl.ds(0, 32), :],
            t2_hbm.at[pl.ds(T2_ROWS - 32, 32), :],
            outs[0],
        ).start()
        pltpu.make_async_copy(
            v0.at[pl.ds(0, 32), :],
            t2_hbm.at[pl.ds(T2_ROWS - 32, 32), :],
            outs[0],
        ).wait()


@functools.partial(
    pl.kernel,
    mesh=_mesh,
    out_type=jax.ShapeDtypeStruct((HIST_LEN, EMBEDDING_DIM, BATCH), jnp.float32),
    scratch_types=(
        [pltpu.VMEM((HIST_LEN, BPW), jnp.int32)]
        + [pltpu.VMEM((128, 128), jnp.float32) for _ in range(2)]
        + [pltpu.VMEM((64, 128), jnp.float32) for _ in range(2)]
        + [pltpu.VMEM((128,), jnp.int32) for _ in range(4)]
        + [pltpu.SemaphoreType.DMA for _ in range(4)]
    ),
    compiler_params=pltpu.CompilerParams(needs_layout_passes=False),
)
def _gather(idxt_hbm, t2_hbm, q_hbm, iv, g0, g1, s0, s1,
            p0, p1, r0, r1, in0, in1, out0, out1):
    w = _worker_id()
    b0w = w * BPW
    gb, sb = (g0, g1), (s0, s1)
    pb, rb = (p0, p1), (r0, r1)
    ins, outs = (in0, in1), (out0, out1)

    pltpu.sync_copy(idxt_hbm.at[:, pl.ds(pl.multiple_of(b0w, 128), BPW)], iv)

    def prep(ci, b):
        h = ci >> 2
        cb = ci & 3
        for m in range(8):
            v = iv[h, pl.ds(cb * 128 + 16 * m, 16)]
            pb[b][pl.ds(16 * m, 16)] = v >> 1
            rb[b][pl.ds(16 * m, 16)] = (v & 1) * 64

    def in_start(b):
        pltpu.make_async_copy(t2_hbm.at[pb[b]], gb[b], ins[b]).start()

    def in_wait(b):
        pltpu.make_async_copy(t2_hbm.at[pb[b]], gb[b], ins[b]).wait()

    def out_start(ci, b):
        h = ci >> 2
        cb = ci & 3
        off = pl.multiple_of(b0w + cb * 128, 128)
        pltpu.make_async_copy(
            sb[b], q_hbm.at[h, :, pl.ds(off, 128)], outs[b]
        ).start()

    def out_wait(b):
        pltpu.make_async_copy(
            sb[b], q_hbm.at[0, :, pl.ds(0, 128)], outs[b]
        ).wait()

    def transpose_sel(b):
        for m in range(8):
            c16 = _iota16() + 16 * m
            rbase = rb[b][pl.ds(16 * m, 16)]

            @plsc.parallel_loop(0, 64, unroll=16)
            def _(j0):
                t = _iota16() + j0
                jl = t - jnp.where(t >= 64, 64, 0)
                vals = plsc.load_gather(gb[b], [c16, rbase + jl])
                plsc.store_scatter(sb[b], [jl, c16], vals)

    prep(0, 0)
    in_start(0)
    prep(1, 1)
    in_start(1)

    def group(g, carry):
        for b in range(2):
            ci = 2 * g + b
            in_wait(b)

            @pl.when(g >= 1)
            def _():
                out_wait(b)

            transpose_sel(b)
            out_start(ci, b)

            @pl.when(g < NCHB // 2 - 1)
            def _():
                prep(ci + 2, b)
                in_start(b)

        return carry

    lax.fori_loop(0, NCHB // 2, group, 0)
    out_wait(0)
    out_wait(1)


def kernel(input, table):
    tabt = table.T
    idxt = input.T.astype(jnp.int32)
    tail2 = table[NUM_EMBEDDINGS - 64:].reshape(32, 128)
    t2 = _repack(tabt, tail2)
    q = _gather(idxt, t2)
    return jnp.transpose(q, (2, 0, 1))


# speedup vs baseline: 1.0916x; 1.0916x over previous
import functools

import jax
import jax.numpy as jnp
from jax import lax
from jax.experimental import pallas as pl
from jax.experimental.pallas import tpu as pltpu
from jax.experimental.pallas import tpu_sc as plsc

NUM_EMBEDDINGS = 1000000
EMBEDDING_DIM = 64
BATCH = 16384
HIST_LEN = 50

NC = 2
NS = 16
NW = NC * NS

T2_ROWS = NUM_EMBEDDINGS // 2
KA_FULL = NUM_EMBEDDINGS // 128
KA_MAIN = (KA_FULL // NW) & ~1
KA_REST = KA_FULL - KA_MAIN * NW
BPW = BATCH // NW
NCHB = HIST_LEN * (BPW // 128)

_mesh = plsc.VectorSubcoreMesh(core_axis_name="c", subcore_axis_name="s")


def _iota16():
    return lax.iota(jnp.int32, 16)


def _worker_id():
    return lax.axis_index("s") * NC + lax.axis_index("c")


def _transpose_pairs(v_ref, s_ref, rows):
    @plsc.parallel_loop(0, rows, unroll=8)
    def _(r0):
        t = _iota16() + r0
        rl = t - jnp.where(t >= rows, rows, 0)
        for m in range(4):
            q16 = _iota16() + 16 * m
            vals0 = plsc.load_gather(v_ref, [q16, 2 * rl])
            plsc.store_scatter(s_ref, [rl, q16], vals0)
            vals1 = plsc.load_gather(v_ref, [q16, 2 * rl + 1])
            plsc.store_scatter(s_ref, [rl, q16 + 64], vals1)


@functools.partial(
    pl.kernel,
    mesh=_mesh,
    out_type=jax.ShapeDtypeStruct((T2_ROWS, 128), jnp.float32),
    scratch_types=(
        [pltpu.VMEM((64, 128), jnp.float32) for _ in range(4)]
        + [pltpu.SemaphoreType.DMA for _ in range(4)]
    ),
    compiler_params=pltpu.CompilerParams(needs_layout_passes=False),
)
def _repack(tabt_hbm, tail2_hbm, t2_hbm, v0, v1, s0, s1, in0, in1, out0, out1):
    w = _worker_id()
    k0 = w * KA_MAIN
    vb, sb, ins, outs = (v0, v1), (s0, s1), (in0, in1), (out0, out1)

    def in_start(k, b):
        off = pl.multiple_of(k * 128, 128)
        pltpu.make_async_copy(
            tabt_hbm.at[:, pl.ds(off, 128)], vb[b], ins[b]
        ).start()

    def in_wait(b):
        pltpu.make_async_copy(
            tabt_hbm.at[:, pl.ds(0, 128)], vb[b], ins[b]
        ).wait()

    def out_start(k, b):
        off = pl.multiple_of(k * 64, 64)
        pltpu.make_async_copy(
            sb[b], t2_hbm.at[pl.ds(off, 64), :], outs[b]
        ).start()

    def out_wait(b):
        pltpu.make_async_copy(
            sb[b], t2_hbm.at[pl.ds(0, 64), :], outs[b]
        ).wait()

    in_start(k0, 0)
    in_start(k0 + 1, 1)

    def group(g, carry):
        for b in range(2):
            k = k0 + 2 * g + b
            in_wait(b)

            @pl.when(g >= 1)
            def _():
                out_wait(b)

            _transpose_pairs(vb[b], sb[b], 64)
            out_start(k, b)

            @pl.when(g < KA_MAIN // 2 - 1)
            def _():
                in_start(k + 2, b)

        return carry

    lax.fori_loop(0, KA_MAIN // 2, group, 0)
    out_wait(0)
    out_wait(1)

    @pl.when(w < KA_REST)
    def _():
        k = KA_MAIN * NW + w
        in_start(k, 0)
        in_wait(0)
        _transpose_pairs(vb[0], sb[0], 64)
        out_start(k, 0)
        out_wait(0)

    @pl.when(w == KA_REST)
    def _():
        pltpu.make_async_copy(tail2_hbm, v0.at[pl.ds(0, 32), :], ins[0]).start()
        pltpu.make_async_copy(tail2_hbm, v0.at[pl.ds(0, 32), :], ins[0]).wait()
        pltpu.make_async_copy(
            v0.at[pl.ds(0, 32), :],
            t2_hbm.at[pl.ds(T2_ROWS - 32, 32), :],
            outs[0],
        ).start()
        pltpu.make_async_copy(
            v0.at[pl.ds(0, 32), :],
            t2_hbm.at[pl.ds(T2_ROWS - 32, 32), :],
            outs[0],
        ).wait()


@functools.partial(
    pl.kernel,
    mesh=_mesh,
    out_type=jax.ShapeDtypeStruct((HIST_LEN, EMBEDDING_DIM, BATCH), jnp.float32),
    scratch_types=(
        [pltpu.VMEM((HIST_LEN, BPW), jnp.int32)]
        + [pltpu.VMEM((128, 128), jnp.float32) for _ in range(2)]
        + [pltpu.VMEM((64, 128), jnp.float32) for _ in range(2)]
        + [pltpu.VMEM((128,), jnp.int32) for _ in range(4)]
        + [pltpu.SemaphoreType.DMA for _ in range(4)]
    ),
    compiler_params=pltpu.CompilerParams(needs_layout_passes=False),
)
def _gather(idxt_hbm, t2_hbm, q_hbm, iv, g0, g1, s0, s1,
            p0, p1, r0, r1, in0, in1, out0, out1):
    w = _worker_id()
    b0w = w * BPW
    gb, sb = (g0, g1), (s0, s1)
    pb, rb = (p0, p1), (r0, r1)
    ins, outs = (in0, in1), (out0, out1)

    pltpu.sync_copy(idxt_hbm.at[:, pl.ds(pl.multiple_of(b0w, 128), BPW)], iv)

    def prep(ci, b):
        h = ci >> 2
        cb = ci & 3
        for m in range(8):
            v = iv[h, pl.ds(cb * 128 + 16 * m, 16)]
            pb[b][pl.ds(16 * m, 16)] = v >> 1
            rb[b][pl.ds(16 * m, 16)] = (v & 1) * 64

    def in_start(b):
        pltpu.make_async_copy(t2_hbm.at[pb[b]], gb[b], ins[b]).start()

    def in_wait(b):
        pltpu.make_async_copy(t2_hbm.at[pb[b]], gb[b], ins[b]).wait()

    def out_start(ci, b):
        h = ci >> 2
        cb = ci & 3
        off = pl.multiple_of(b0w + cb * 128, 128)
        pltpu.make_async_copy(
            sb[b], q_hbm.at[h, :, pl.ds(off, 128)], outs[b]
        ).start()

    def out_wait(b):
        pltpu.make_async_copy(
            sb[b], q_hbm.at[0, :, pl.ds(0, 128)], outs[b]
        ).wait()

    def transpose_sel(b):
        for m in range(8):
            c16 = _iota16() + 16 * m
            rbase = rb[b][pl.ds(16 * m, 16)]

            @plsc.parallel_loop(0, 64, unroll=8)
            def _(j0):
                t = _iota16() + j0
                jl = t - jnp.where(t >= 64, 64, 0)
                vals = plsc.load_gather(gb[b], [c16, rbase + jl])
                plsc.store_scatter(sb[b], [jl, c16], vals)

    prep(0, 0)
    in_start(0)
    prep(1, 1)
    in_start(1)

    def group(g, carry):
        for b in range(2):
            ci = 2 * g + b
            in_wait(b)

            @pl.when(g >= 1)
            def _():
                out_wait(b)

            transpose_sel(b)
            out_start(ci, b)

            @pl.when(g < NCHB // 2 - 1)
            def _():
                prep(ci + 2, b)
                in_start(b)

        return carry

    lax.fori_loop(0, NCHB // 2, group, 0)
    out_wait(0)
    out_wait(1)


def kernel(input, table):
    tabt = table.T
    idxt = input.T.astype(jnp.int32)
    tail2 = table[NUM_EMBEDDINGS - 64:].reshape(32, 128)
    t2 = _repack(tabt, tail2)
    q = _gather(idxt, t2)
    return jnp.transpose(q, (2, 0, 1))
